# baseline (speedup 1.0000x reference)
import functools

import jax
import jax.numpy as jnp
from jax import lax
from jax.experimental import pallas as pl
from jax.experimental.pallas import tpu as pltpu
from jax.experimental.pallas import tpu_sc as plsc

T = 4096
H = 2048
I = 1408
E = 8
K = 2
A = T * K
M = 512
MSH = 9
NB = 24
S = NB * M
NW = 32
CHUNK = A // NW
IS_ = I
NG = 16
GRP = CHUNK // NG
BT = 512


def _lane():
    return lax.iota(jnp.int32, 16)


def _cast_body(w_ref, o_ref):
    o_ref[...] = w_ref[...].astype(jnp.bfloat16)


def _cast_bf16(w, second, minor):
    return pl.pallas_call(
        _cast_body,
        grid=(E, 2),
        in_specs=[pl.BlockSpec((1, second // 2, minor), lambda e, i: (e, i, 0))],
        out_specs=pl.BlockSpec((1, second // 2, minor), lambda e, i: (e, i, 0)),
        out_shape=jax.ShapeDtypeStruct(w.shape, jnp.bfloat16),
    )(w)


def _cast_bf16_2d(w, second, minor):
    return pl.pallas_call(
        _cast_body,
        grid=(2,),
        in_specs=[pl.BlockSpec((second // 2, minor), lambda i: (i, 0))],
        out_specs=pl.BlockSpec((second // 2, minor), lambda i: (i, 0)),
        out_shape=jax.ShapeDtypeStruct(w.shape, jnp.bfloat16),
    )(w)


def _router_shared_body(x_ref, rw_ref, bias_ref, g_ref, u_ref, d_ref,
                        out_ref, idx_ref, w_ref):
    x = x_ref[...]
    logits = jnp.dot(x, rw_ref[...].T, preferred_element_type=jnp.float32)
    logits = logits + bias_ref[...]
    scores = jax.nn.sigmoid(logits)
    iota = lax.broadcasted_iota(jnp.int32, scores.shape, 1)
    w1 = jnp.max(scores, axis=-1, keepdims=True)
    i1 = jnp.argmax(scores, axis=-1)[:, None]
    masked = jnp.where(iota == i1, -jnp.inf, scores)
    w2 = jnp.max(masked, axis=-1, keepdims=True)
    i2 = jnp.argmax(masked, axis=-1)[:, None]
    denom = w1 + w2 + 1e-8
    idx_ref[...] = jnp.concatenate([i1, i2], axis=1)
    w_ref[...] = jnp.concatenate([w1 / denom, w2 / denom], axis=1)

    xb = x.astype(jnp.bfloat16)
    g = jnp.dot(xb, g_ref[...].T, preferred_element_type=jnp.float32)
    u = jnp.dot(xb, u_ref[...].T, preferred_element_type=jnp.float32)
    h = (jax.nn.silu(g) * u).astype(jnp.bfloat16)
    out_ref[...] = jnp.dot(h, d_ref[...].T, preferred_element_type=jnp.float32)


def _dispatch_body(eids_hbm, xb_hbm, xs_hbm, pos_hbm, btbl_hbm,
                   eids_v, posb, tokb, rows0, rows1, btbl_v,
                   hist_all, hist_pre, runh,
                   gsem0, gsem1, ssem0, ssem1):
    _LANE = _lane()
    wid = lax.axis_index("c") * 16 + lax.axis_index("s")
    my_vreg0 = wid * (CHUNK // 16)

    pltpu.sync_copy(eids_hbm, eids_v)

    lane_eq = [(_LANE == e) for e in range(E)]
    zeros16 = jnp.zeros((16,), jnp.int32)
    ones16 = jnp.full((16,), 1, jnp.int32)
    hist_all[...] = zeros16
    hist_pre[...] = zeros16

    def scan_step(i, carry):
        v = eids_v[pl.ds(i * 16, 16)]
        plsc.addupdate_scatter(hist_all, [v], ones16)
        pre = jnp.where(i < my_vreg0, 1, 0)
        plsc.addupdate_scatter(hist_pre, [v], ones16 * pre)
        return carry

    lax.fori_loop(0, A // 16, scan_step, 0)
    cnt_all = hist_all[...]
    cnt_pre = hist_pre[...]

    nb = (cnt_all + (M - 1)) >> MSH
    nbc_incl = plsc.cumsum(nb)
    pstart = (nbc_incl - nb) * M

    runh[...] = pstart + cnt_pre
    for j in range(CHUNK // 16):
        v = eids_v[pl.ds((my_vreg0 + j) * 16, 16)]
        base = plsc.load_gather(runh, [v])
        excl = zeros16
        for e in range(E):
            m = (v == e).astype(jnp.int32)
            cs = plsc.cumsum(m)
            excl = excl + (cs - m) * m
        plsc.addupdate_scatter(runh, [v], ones16)
        posb[j, :] = base + excl
        tokb[j, :] = (wid * CHUNK + j * 16 + _LANE) >> 1

    @pl.when(wid == 0)
    def _():
        nbc_s = [jnp.sum(jnp.where(lane_eq[e], nbc_incl, 0)) for e in range(E)]
        total = nbc_s[E - 1]
        for r in range(3):
            b = _LANE + r * 16
            acc = jnp.zeros((16,), jnp.int32)
            for e in range(E):
                lo = nbc_s[e - 1] if e > 0 else 0
                acc = acc + e * ((b >= lo) & (b < nbc_s[e])).astype(jnp.int32)
            btbl_v[pl.ds(r * 16, 16)] = jnp.where(b < total, acc, 0)
        pltpu.sync_copy(btbl_v, btbl_hbm)

    pltpu.sync_copy(posb, pos_hbm.at[wid])

    bufs = (rows0, rows1)
    gsems = (gsem0, gsem1)
    ssems = (ssem0, ssem1)
    scat = [None, None]

    def gather(g):
        return pltpu.async_copy(xb_hbm.at[tokb.at[g]], bufs[g % 2],
                                gsems[g % 2])

    cg = gather(0)
    for g in range(NG):
        cg.wait()
        scat[g % 2] = pltpu.async_copy(bufs[g % 2], xs_hbm.at[posb.at[g]],
                                       ssems[g % 2])
        if g + 1 < NG:
            if scat[(g + 1) % 2] is not None:
                scat[(g + 1) % 2].wait()
            cg = gather(g + 1)
    scat[0].wait()
    scat[1].wait()


def _ffn_block_body(s_ref, xs_ref, g_ref, u_ref, d_ref, y_ref):
    xb = xs_ref[...].astype(jnp.bfloat16)
    g = jnp.dot(xb, g_ref[0].T, preferred_element_type=jnp.float32)
    u = jnp.dot(xb, u_ref[0].T, preferred_element_type=jnp.float32)
    h = (jax.nn.silu(g) * u).astype(jnp.bfloat16)
    y_ref[...] = jnp.dot(h, d_ref[0].T, preferred_element_type=jnp.float32)


def _combine_body(init_hbm, y_hbm, pos_hbm, w_hbm, out_hbm,
                  posb, wb, y0, y1, shbuf, gsem0, gsem1, sem):
    _LANE = _lane()
    wid = lax.axis_index("c") * 16 + lax.axis_index("s")
    pltpu.sync_copy(pos_hbm.at[wid], posb)
    pltpu.sync_copy(w_hbm.at[wid], wb)
    ybufs = (y0, y1)
    gsems = (gsem0, gsem1)
    TPG = GRP // 2

    def gather(g):
        return pltpu.async_copy(y_hbm.at[posb.at[g]], ybufs[g % 2],
                                gsems[g % 2])

    cg = gather(0)
    for g in range(NG):
        tok_base = wid * (T // NW) + g * TPG
        pltpu.sync_copy(init_hbm.at[pl.ds(tok_base, TPG)], shbuf)
        wrow = wb[g, :]
        ws = []
        for a in range(GRP):
            ws.append(jnp.sum(jnp.where(_LANE == a, wrow, 0.0)))
        cg.wait()
        yb = ybufs[g % 2]
        if g + 1 < NG:
            cg = gather(g + 1)

        def vbody(i, _, yb=yb, ws=ws):
            sl = pl.ds(i * 16, 16)
            for t in range(TPG):
                shbuf[t, sl] = (shbuf[t, sl] + ws[2 * t] * yb[2 * t, sl]
                                + ws[2 * t + 1] * yb[2 * t + 1, sl])
            return 0

        lax.fori_loop(0, H // 16, vbody, 0)
        pltpu.sync_copy(shbuf, out_hbm.at[pl.ds(tok_base, TPG)])


def kernel(x, router_w, lb_bias, gate_w, up_w, down_w, sh_gate_w, sh_up_w, sh_down_w):
    bias2d = lb_bias.reshape(1, E)

    grid = (T // BT,)
    out_init, topk_idx, topk_w = pl.pallas_call(
        _router_shared_body,
        grid=grid,
        in_specs=[
            pl.BlockSpec((BT, H), lambda t: (t, 0)),
            pl.BlockSpec((E, H), lambda t: (0, 0)),
            pl.BlockSpec((1, E), lambda t: (0, 0)),
            pl.BlockSpec((I, H), lambda t: (0, 0)),
            pl.BlockSpec((I, H), lambda t: (0, 0)),
            pl.BlockSpec((H, I), lambda t: (0, 0)),
        ],
        out_specs=[
            pl.BlockSpec((BT, H), lambda t: (t, 0)),
            pl.BlockSpec((BT, K), lambda t: (t, 0)),
            pl.BlockSpec((BT, K), lambda t: (t, 0)),
        ],
        out_shape=[
            jax.ShapeDtypeStruct((T, H), jnp.float32),
            jax.ShapeDtypeStruct((T, K), jnp.int32),
            jax.ShapeDtypeStruct((T, K), jnp.float32),
        ],
    )(x, router_w, bias2d,
      _cast_bf16_2d(sh_gate_w, IS_, H), _cast_bf16_2d(sh_up_w, IS_, H),
      _cast_bf16_2d(sh_down_w, H, IS_))

    gate_b = _cast_bf16(gate_w, I, H)
    up_b = _cast_bf16(up_w, I, H)
    down_b = _cast_bf16(down_w, H, I)

    eids = topk_idx.reshape(A)
    w3d = topk_w.reshape(NW, NG, GRP)

    mesh = plsc.VectorSubcoreMesh(core_axis_name="c", subcore_axis_name="s")
    sc_params = pltpu.CompilerParams(needs_layout_passes=False)
    dispatch = functools.partial(
        pl.kernel,
        mesh=mesh,
        compiler_params=sc_params,
        out_type=[
            jax.ShapeDtypeStruct((S, H), jnp.float32),
            jax.ShapeDtypeStruct((NW, NG, GRP), jnp.int32),
            jax.ShapeDtypeStruct((48,), jnp.int32),
        ],
        scratch_types=[
            pltpu.VMEM((A,), jnp.int32),
            pltpu.VMEM((NG, GRP), jnp.int32),
            pltpu.VMEM((NG, GRP), jnp.int32),
            pltpu.VMEM((GRP, H), jnp.float32),
            pltpu.VMEM((GRP, H), jnp.float32),
            pltpu.VMEM((48,), jnp.int32),
            pltpu.VMEM((16,), jnp.int32),
            pltpu.VMEM((16,), jnp.int32),
            pltpu.VMEM((16,), jnp.int32),
            pltpu.SemaphoreType.DMA,
            pltpu.SemaphoreType.DMA,
            pltpu.SemaphoreType.DMA,
            pltpu.SemaphoreType.DMA,
        ],
    )(_dispatch_body)
    x_sorted, pos, btbl = dispatch(eids, x)

    grid_spec = pltpu.PrefetchScalarGridSpec(
        num_scalar_prefetch=1,
        grid=(NB,),
        in_specs=[
            pl.BlockSpec((M, H), lambda b, s: (b, 0)),
            pl.BlockSpec((1, I, H), lambda b, s: (s[b], 0, 0)),
            pl.BlockSpec((1, I, H), lambda b, s: (s[b], 0, 0)),
            pl.BlockSpec((1, H, I), lambda b, s: (s[b], 0, 0)),
        ],
        out_specs=pl.BlockSpec((M, H), lambda b, s: (b, 0)),
    )
    y_sorted = pl.pallas_call(
        _ffn_block_body,
        grid_spec=grid_spec,
        out_shape=jax.ShapeDtypeStruct((S, H), jnp.float32),
    )(btbl, x_sorted, gate_b, up_b, down_b)

    combine = functools.partial(
        pl.kernel,
        mesh=mesh,
        compiler_params=sc_params,
        out_type=jax.ShapeDtypeStruct((T, H), jnp.float32),
        scratch_types=[
            pltpu.VMEM((NG, GRP), jnp.int32),
            pltpu.VMEM((NG, GRP), jnp.float32),
            pltpu.VMEM((GRP, H), jnp.float32),
            pltpu.VMEM((GRP, H), jnp.float32),
            pltpu.VMEM((GRP // 2, H), jnp.float32),
            pltpu.SemaphoreType.DMA,
            pltpu.SemaphoreType.DMA,
            pltpu.SemaphoreType.DMA,
        ],
    )(_combine_body)
    return combine(out_init, y_sorted, pos, w3d)


# speedup vs baseline: 1.0275x; 1.0275x over previous
import functools

import jax
import jax.numpy as jnp
from jax import lax
from jax.experimental import pallas as pl
from jax.experimental.pallas import tpu as pltpu
from jax.experimental.pallas import tpu_sc as plsc

T = 4096
H = 2048
I = 1408
E = 8
K = 2
A = T * K
M = 256
MSH = 8
NB = 40
S = NB * M
NW = 32
CHUNK = A // NW
IS_ = I
NG = 16
GRP = CHUNK // NG
BT = 512


def _lane():
    return lax.iota(jnp.int32, 16)


def _cast_body(w_ref, o_ref):
    o_ref[...] = w_ref[...].astype(jnp.bfloat16)


def _cast3_body(g_ref, u_ref, d_ref, go_ref, uo_ref, do_ref):
    go_ref[...] = g_ref[...].astype(jnp.bfloat16)
    uo_ref[...] = u_ref[...].astype(jnp.bfloat16)
    do_ref[...] = d_ref[...].astype(jnp.bfloat16)


def _cast3_bf16(g, u, d):
    spec_ih = pl.BlockSpec((1, I // 2, H), lambda e, i: (e, i, 0))
    spec_hi = pl.BlockSpec((1, H // 2, I), lambda e, i: (e, i, 0))
    return pl.pallas_call(
        _cast3_body,
        grid=(E, 2),
        in_specs=[spec_ih, spec_ih, spec_hi],
        out_specs=[spec_ih, spec_ih, spec_hi],
        out_shape=[
            jax.ShapeDtypeStruct(g.shape, jnp.bfloat16),
            jax.ShapeDtypeStruct(u.shape, jnp.bfloat16),
            jax.ShapeDtypeStruct(d.shape, jnp.bfloat16),
        ],
    )(g, u, d)


def _cast_bf16_2d(w, second, minor):
    return pl.pallas_call(
        _cast_body,
        grid=(2,),
        in_specs=[pl.BlockSpec((second // 2, minor), lambda i: (i, 0))],
        out_specs=pl.BlockSpec((second // 2, minor), lambda i: (i, 0)),
        out_shape=jax.ShapeDtypeStruct(w.shape, jnp.bfloat16),
    )(w)


def _router_shared_body(x_ref, rw_ref, bias_ref, g_ref, u_ref, d_ref,
                        out_ref, idx_ref, w_ref):
    x = x_ref[...]
    logits = jnp.dot(x, rw_ref[...].T, preferred_element_type=jnp.float32)
    logits = logits + bias_ref[...]
    scores = jax.nn.sigmoid(logits)
    iota = lax.broadcasted_iota(jnp.int32, scores.shape, 1)
    w1 = jnp.max(scores, axis=-1, keepdims=True)
    i1 = jnp.argmax(scores, axis=-1)[:, None]
    masked = jnp.where(iota == i1, -jnp.inf, scores)
    w2 = jnp.max(masked, axis=-1, keepdims=True)
    i2 = jnp.argmax(masked, axis=-1)[:, None]
    denom = w1 + w2 + 1e-8
    idx_ref[...] = jnp.concatenate([i1, i2], axis=1)
    w_ref[...] = jnp.concatenate([w1 / denom, w2 / denom], axis=1)

    xb = x.astype(jnp.bfloat16)
    g = jnp.dot(xb, g_ref[...].T, preferred_element_type=jnp.float32)
    u = jnp.dot(xb, u_ref[...].T, preferred_element_type=jnp.float32)
    h = (jax.nn.silu(g) * u).astype(jnp.bfloat16)
    out_ref[...] = jnp.dot(h, d_ref[...].T, preferred_element_type=jnp.float32)


def _dispatch_body(eids_hbm, xb_hbm, xs_hbm, pos_hbm, btbl_hbm,
                   eids_v, posb, tokb, rows0, rows1, btbl_v,
                   hist_all, hist_pre, runh,
                   gsem0, gsem1, ssem0, ssem1):
    _LANE = _lane()
    wid = lax.axis_index("c") * 16 + lax.axis_index("s")
    my_vreg0 = wid * (CHUNK // 16)

    pltpu.sync_copy(eids_hbm, eids_v)

    lane_eq = [(_LANE == e) for e in range(E)]
    zeros16 = jnp.zeros((16,), jnp.int32)
    ones16 = jnp.full((16,), 1, jnp.int32)
    hist_all[...] = zeros16
    hist_pre[...] = zeros16

    def scan_step(i, carry):
        v = eids_v[pl.ds(i * 16, 16)]
        plsc.addupdate_scatter(hist_all, [v], ones16)
        pre = jnp.where(i < my_vreg0, 1, 0)
        plsc.addupdate_scatter(hist_pre, [v], ones16 * pre)
        return carry

    lax.fori_loop(0, A // 16, scan_step, 0)
    cnt_all = hist_all[...]
    cnt_pre = hist_pre[...]

    nb = (cnt_all + (M - 1)) >> MSH
    nbc_incl = plsc.cumsum(nb)
    pstart = (nbc_incl - nb) * M

    runh[...] = pstart + cnt_pre
    for j in range(CHUNK // 16):
        v = eids_v[pl.ds((my_vreg0 + j) * 16, 16)]
        base = plsc.load_gather(runh, [v])
        excl = zeros16
        for e in range(E):
            m = (v == e).astype(jnp.int32)
            cs = plsc.cumsum(m)
            excl = excl + (cs - m) * m
        plsc.addupdate_scatter(runh, [v], ones16)
        posb[j, :] = base + excl
        tokb[j, :] = (wid * CHUNK + j * 16 + _LANE) >> 1

    @pl.when(wid == 0)
    def _():
        nbc_s = [jnp.sum(jnp.where(lane_eq[e], nbc_incl, 0)) for e in range(E)]
        total = nbc_s[E - 1]
        for r in range(3):
            b = _LANE + r * 16
            acc = jnp.zeros((16,), jnp.int32)
            for e in range(E):
                lo = nbc_s[e - 1] if e > 0 else 0
                acc = acc + e * ((b >= lo) & (b < nbc_s[e])).astype(jnp.int32)
            btbl_v[pl.ds(r * 16, 16)] = jnp.where(b < total, acc, 0)
        pltpu.sync_copy(btbl_v, btbl_hbm)

    pltpu.sync_copy(posb, pos_hbm.at[wid])

    bufs = (rows0, rows1)
    gsems = (gsem0, gsem1)
    ssems = (ssem0, ssem1)
    scat = [None, None]

    def gather(g):
        return pltpu.async_copy(xb_hbm.at[tokb.at[g]], bufs[g % 2],
                                gsems[g % 2])

    cg = gather(0)
    for g in range(NG):
        cg.wait()
        scat[g % 2] = pltpu.async_copy(bufs[g % 2], xs_hbm.at[posb.at[g]],
                                       ssems[g % 2])
        if g + 1 < NG:
            if scat[(g + 1) % 2] is not None:
                scat[(g + 1) % 2].wait()
            cg = gather(g + 1)
    scat[0].wait()
    scat[1].wait()


def _ffn_block_body(s_ref, xs_ref, g_ref, u_ref, d_ref, y_ref):
    xb = xs_ref[...].astype(jnp.bfloat16)
    g = jnp.dot(xb, g_ref[0].T, preferred_element_type=jnp.float32)
    u = jnp.dot(xb, u_ref[0].T, preferred_element_type=jnp.float32)
    h = (jax.nn.silu(g) * u).astype(jnp.bfloat16)
    y_ref[...] = jnp.dot(h, d_ref[0].T, preferred_element_type=jnp.float32)


def _combine_body(init_hbm, y_hbm, pos_hbm, w_hbm, out_hbm,
                  posb, wb, y0, y1, shbuf, gsem0, gsem1, sem):
    _LANE = _lane()
    wid = lax.axis_index("c") * 16 + lax.axis_index("s")
    pltpu.sync_copy(pos_hbm.at[wid], posb)
    pltpu.sync_copy(w_hbm.at[wid], wb)
    ybufs = (y0, y1)
    gsems = (gsem0, gsem1)
    TPG = GRP // 2

    def gather(g):
        return pltpu.async_copy(y_hbm.at[posb.at[g]], ybufs[g % 2],
                                gsems[g % 2])

    cg = gather(0)
    for g in range(NG):
        tok_base = wid * (T // NW) + g * TPG
        pltpu.sync_copy(init_hbm.at[pl.ds(tok_base, TPG)], shbuf)
        wrow = wb[g, :]
        ws = []
        for a in range(GRP):
            ws.append(jnp.sum(jnp.where(_LANE == a, wrow, 0.0)))
        cg.wait()
        yb = ybufs[g % 2]
        if g + 1 < NG:
            cg = gather(g + 1)

        def vbody(i, _, yb=yb, ws=ws):
            sl = pl.ds(i * 16, 16)
            for t in range(TPG):
                shbuf[t, sl] = (shbuf[t, sl] + ws[2 * t] * yb[2 * t, sl]
                                + ws[2 * t + 1] * yb[2 * t + 1, sl])
            return 0

        lax.fori_loop(0, H // 16, vbody, 0)
        pltpu.sync_copy(shbuf, out_hbm.at[pl.ds(tok_base, TPG)])


def kernel(x, router_w, lb_bias, gate_w, up_w, down_w, sh_gate_w, sh_up_w, sh_down_w):
    bias2d = lb_bias.reshape(1, E)

    grid = (T // BT,)
    out_init, topk_idx, topk_w = pl.pallas_call(
        _router_shared_body,
        grid=grid,
        in_specs=[
            pl.BlockSpec((BT, H), lambda t: (t, 0)),
            pl.BlockSpec((E, H), lambda t: (0, 0)),
            pl.BlockSpec((1, E), lambda t: (0, 0)),
            pl.BlockSpec((I, H), lambda t: (0, 0)),
            pl.BlockSpec((I, H), lambda t: (0, 0)),
            pl.BlockSpec((H, I), lambda t: (0, 0)),
        ],
        out_specs=[
            pl.BlockSpec((BT, H), lambda t: (t, 0)),
            pl.BlockSpec((BT, K), lambda t: (t, 0)),
            pl.BlockSpec((BT, K), lambda t: (t, 0)),
        ],
        out_shape=[
            jax.ShapeDtypeStruct((T, H), jnp.float32),
            jax.ShapeDtypeStruct((T, K), jnp.int32),
            jax.ShapeDtypeStruct((T, K), jnp.float32),
        ],
    )(x, router_w, bias2d,
      _cast_bf16_2d(sh_gate_w, IS_, H), _cast_bf16_2d(sh_up_w, IS_, H),
      _cast_bf16_2d(sh_down_w, H, IS_))

    gate_b, up_b, down_b = _cast3_bf16(gate_w, up_w, down_w)

    eids = topk_idx.reshape(A)
    w3d = topk_w.reshape(NW, NG, GRP)

    mesh = plsc.VectorSubcoreMesh(core_axis_name="c", subcore_axis_name="s")
    sc_params = pltpu.CompilerParams(needs_layout_passes=False)
    dispatch = functools.partial(
        pl.kernel,
        mesh=mesh,
        compiler_params=sc_params,
        out_type=[
            jax.ShapeDtypeStruct((S, H), jnp.float32),
            jax.ShapeDtypeStruct((NW, NG, GRP), jnp.int32),
            jax.ShapeDtypeStruct((48,), jnp.int32),
        ],
        scratch_types=[
            pltpu.VMEM((A,), jnp.int32),
            pltpu.VMEM((NG, GRP), jnp.int32),
            pltpu.VMEM((NG, GRP), jnp.int32),
            pltpu.VMEM((GRP, H), jnp.float32),
            pltpu.VMEM((GRP, H), jnp.float32),
            pltpu.VMEM((48,), jnp.int32),
            pltpu.VMEM((16,), jnp.int32),
            pltpu.VMEM((16,), jnp.int32),
            pltpu.VMEM((16,), jnp.int32),
            pltpu.SemaphoreType.DMA,
            pltpu.SemaphoreType.DMA,
            pltpu.SemaphoreType.DMA,
            pltpu.SemaphoreType.DMA,
        ],
    )(_dispatch_body)
    x_sorted, pos, btbl = dispatch(eids, x)

    grid_spec = pltpu.PrefetchScalarGridSpec(
        num_scalar_prefetch=1,
        grid=(NB,),
        in_specs=[
            pl.BlockSpec((M, H), lambda b, s: (b, 0)),
            pl.BlockSpec((1, I, H), lambda b, s: (s[b], 0, 0)),
            pl.BlockSpec((1, I, H), lambda b, s: (s[b], 0, 0)),
            pl.BlockSpec((1, H, I), lambda b, s: (s[b], 0, 0)),
        ],
        out_specs=pl.BlockSpec((M, H), lambda b, s: (b, 0)),
    )
    y_sorted = pl.pallas_call(
        _ffn_block_body,
        grid_spec=grid_spec,
        out_shape=jax.ShapeDtypeStruct((S, H), jnp.float32),
    )(btbl, x_sorted, gate_b, up_b, down_b)

    combine = functools.partial(
        pl.kernel,
        mesh=mesh,
        compiler_params=sc_params,
        out_type=jax.ShapeDtypeStruct((T, H), jnp.float32),
        scratch_types=[
            pltpu.VMEM((NG, GRP), jnp.int32),
            pltpu.VMEM((NG, GRP), jnp.float32),
            pltpu.VMEM((GRP, H), jnp.float32),
            pltpu.VMEM((GRP, H), jnp.float32),
            pltpu.VMEM((GRP // 2, H), jnp.float32),
            pltpu.SemaphoreType.DMA,
            pltpu.SemaphoreType.DMA,
            pltpu.SemaphoreType.DMA,
        ],
    )(_combine_body)
    return combine(out_init, y_sorted, pos, w3d)


# speedup vs baseline: 1.0278x; 1.0002x over previous
import functools

import jax
import jax.numpy as jnp
from jax import lax
from jax.experimental import pallas as pl
from jax.experimental.pallas import tpu as pltpu
from jax.experimental.pallas import tpu_sc as plsc

T = 4096
H = 2048
I = 1408
E = 8
K = 2
A = T * K
M = 256
MSH = 8
NB = 40
S = NB * M
NW = 32
CHUNK = A // NW
IS_ = I
NG = 16
GRP = CHUNK // NG
BT = 1024


def _lane():
    return lax.iota(jnp.int32, 16)


def _cast_body(w_ref, o_ref):
    o_ref[...] = w_ref[...].astype(jnp.bfloat16)


def _cast3_body(g_ref, u_ref, d_ref, go_ref, uo_ref, do_ref):
    go_ref[...] = g_ref[...].astype(jnp.bfloat16)
    uo_ref[...] = u_ref[...].astype(jnp.bfloat16)
    do_ref[...] = d_ref[...].astype(jnp.bfloat16)


def _cast3_bf16(g, u, d):
    spec_ih = pl.BlockSpec((1, I // 2, H), lambda e, i: (e, i, 0))
    spec_hi = pl.BlockSpec((1, H // 2, I), lambda e, i: (e, i, 0))
    return pl.pallas_call(
        _cast3_body,
        grid=(E, 2),
        in_specs=[spec_ih, spec_ih, spec_hi],
        out_specs=[spec_ih, spec_ih, spec_hi],
        out_shape=[
            jax.ShapeDtypeStruct(g.shape, jnp.bfloat16),
            jax.ShapeDtypeStruct(u.shape, jnp.bfloat16),
            jax.ShapeDtypeStruct(d.shape, jnp.bfloat16),
        ],
    )(g, u, d)


def _cast_bf16_2d(w, second, minor):
    return pl.pallas_call(
        _cast_body,
        grid=(2,),
        in_specs=[pl.BlockSpec((second // 2, minor), lambda i: (i, 0))],
        out_specs=pl.BlockSpec((second // 2, minor), lambda i: (i, 0)),
        out_shape=jax.ShapeDtypeStruct(w.shape, jnp.bfloat16),
    )(w)


def _router_shared_body(x_ref, rw_ref, bias_ref, g_ref, u_ref, d_ref,
                        out_ref, idx_ref, w_ref):
    x = x_ref[...]
    logits = jnp.dot(x, rw_ref[...].T, preferred_element_type=jnp.float32)
    logits = logits + bias_ref[...]
    scores = jax.nn.sigmoid(logits)
    iota = lax.broadcasted_iota(jnp.int32, scores.shape, 1)
    w1 = jnp.max(scores, axis=-1, keepdims=True)
    i1 = jnp.argmax(scores, axis=-1)[:, None]
    masked = jnp.where(iota == i1, -jnp.inf, scores)
    w2 = jnp.max(masked, axis=-1, keepdims=True)
    i2 = jnp.argmax(masked, axis=-1)[:, None]
    denom = w1 + w2 + 1e-8
    idx_ref[...] = jnp.concatenate([i1, i2], axis=1)
    w_ref[...] = jnp.concatenate([w1 / denom, w2 / denom], axis=1)

    xb = x.astype(jnp.bfloat16)
    g = jnp.dot(xb, g_ref[...].T, preferred_element_type=jnp.float32)
    u = jnp.dot(xb, u_ref[...].T, preferred_element_type=jnp.float32)
    h = (jax.nn.silu(g) * u).astype(jnp.bfloat16)
    out_ref[...] = jnp.dot(h, d_ref[...].T, preferred_element_type=jnp.float32)


def _dispatch_body(eids_hbm, xb_hbm, xs_hbm, pos_hbm, btbl_hbm,
                   eids_v, posb, tokb, rows0, rows1, btbl_v,
                   hist_all, hist_pre, runh,
                   gsem0, gsem1, ssem0, ssem1):
    _LANE = _lane()
    wid = lax.axis_index("c") * 16 + lax.axis_index("s")
    my_vreg0 = wid * (CHUNK // 16)

    pltpu.sync_copy(eids_hbm, eids_v)

    lane_eq = [(_LANE == e) for e in range(E)]
    zeros16 = jnp.zeros((16,), jnp.int32)
    ones16 = jnp.full((16,), 1, jnp.int32)
    hist_all[...] = zeros16
    hist_pre[...] = zeros16

    def scan_step(i, carry):
        v = eids_v[pl.ds(i * 16, 16)]
        plsc.addupdate_scatter(hist_all, [v], ones16)
        pre = jnp.where(i < my_vreg0, 1, 0)
        plsc.addupdate_scatter(hist_pre, [v], ones16 * pre)
        return carry

    lax.fori_loop(0, A // 16, scan_step, 0)
    cnt_all = hist_all[...]
    cnt_pre = hist_pre[...]

    nb = (cnt_all + (M - 1)) >> MSH
    nbc_incl = plsc.cumsum(nb)
    pstart = (nbc_incl - nb) * M

    runh[...] = pstart + cnt_pre
    for j in range(CHUNK // 16):
        v = eids_v[pl.ds((my_vreg0 + j) * 16, 16)]
        base = plsc.load_gather(runh, [v])
        excl = zeros16
        for e in range(E):
            m = (v == e).astype(jnp.int32)
            cs = plsc.cumsum(m)
            excl = excl + (cs - m) * m
        plsc.addupdate_scatter(runh, [v], ones16)
        posb[j, :] = base + excl
        tokb[j, :] = (wid * CHUNK + j * 16 + _LANE) >> 1

    @pl.when(wid == 0)
    def _():
        nbc_s = [jnp.sum(jnp.where(lane_eq[e], nbc_incl, 0)) for e in range(E)]
        total = nbc_s[E - 1]
        for r in range(3):
            b = _LANE + r * 16
            acc = jnp.zeros((16,), jnp.int32)
            for e in range(E):
                lo = nbc_s[e - 1] if e > 0 else 0
                acc = acc + e * ((b >= lo) & (b < nbc_s[e])).astype(jnp.int32)
            btbl_v[pl.ds(r * 16, 16)] = jnp.where(b < total, acc, 0)
        pltpu.sync_copy(btbl_v, btbl_hbm)

    pltpu.sync_copy(posb, pos_hbm.at[wid])

    bufs = (rows0, rows1)
    gsems = (gsem0, gsem1)
    ssems = (ssem0, ssem1)
    scat = [None, None]

    def gather(g):
        return pltpu.async_copy(xb_hbm.at[tokb.at[g]], bufs[g % 2],
                                gsems[g % 2])

    cg = gather(0)
    for g in range(NG):
        cg.wait()
        scat[g % 2] = pltpu.async_copy(bufs[g % 2], xs_hbm.at[posb.at[g]],
                                       ssems[g % 2])
        if g + 1 < NG:
            if scat[(g + 1) % 2] is not None:
                scat[(g + 1) % 2].wait()
            cg = gather(g + 1)
    scat[0].wait()
    scat[1].wait()


def _ffn_block_body(s_ref, xs_ref, g_ref, u_ref, d_ref, y_ref):
    xb = xs_ref[...].astype(jnp.bfloat16)
    g = jnp.dot(xb, g_ref[0].T, preferred_element_type=jnp.float32)
    u = jnp.dot(xb, u_ref[0].T, preferred_element_type=jnp.float32)
    h = (jax.nn.silu(g) * u).astype(jnp.bfloat16)
    y_ref[...] = jnp.dot(h, d_ref[0].T, preferred_element_type=jnp.float32)


def _combine_body(init_hbm, y_hbm, pos_hbm, w_hbm, out_hbm,
                  posb, wb, y0, y1, shbuf, gsem0, gsem1, sem):
    _LANE = _lane()
    wid = lax.axis_index("c") * 16 + lax.axis_index("s")
    pltpu.sync_copy(pos_hbm.at[wid], posb)
    pltpu.sync_copy(w_hbm.at[wid], wb)
    ybufs = (y0, y1)
    gsems = (gsem0, gsem1)
    TPG = GRP // 2

    def gather(g):
        return pltpu.async_copy(y_hbm.at[posb.at[g]], ybufs[g % 2],
                                gsems[g % 2])

    cg = gather(0)
    for g in range(NG):
        tok_base = wid * (T // NW) + g * TPG
        pltpu.sync_copy(init_hbm.at[pl.ds(tok_base, TPG)], shbuf)
        wrow = wb[g, :]
        ws = []
        for a in range(GRP):
            ws.append(jnp.sum(jnp.where(_LANE == a, wrow, 0.0)))
        cg.wait()
        yb = ybufs[g % 2]
        if g + 1 < NG:
            cg = gather(g + 1)

        def vbody(i, _, yb=yb, ws=ws):
            sl = pl.ds(i * 16, 16)
            for t in range(TPG):
                shbuf[t, sl] = (shbuf[t, sl] + ws[2 * t] * yb[2 * t, sl]
                                + ws[2 * t + 1] * yb[2 * t + 1, sl])
            return 0

        lax.fori_loop(0, H // 16, vbody, 0)
        pltpu.sync_copy(shbuf, out_hbm.at[pl.ds(tok_base, TPG)])


def kernel(x, router_w, lb_bias, gate_w, up_w, down_w, sh_gate_w, sh_up_w, sh_down_w):
    bias2d = lb_bias.reshape(1, E)

    grid = (T // BT,)
    out_init, topk_idx, topk_w = pl.pallas_call(
        _router_shared_body,
        grid=grid,
        in_specs=[
            pl.BlockSpec((BT, H), lambda t: (t, 0)),
            pl.BlockSpec((E, H), lambda t: (0, 0)),
            pl.BlockSpec((1, E), lambda t: (0, 0)),
            pl.BlockSpec((I, H), lambda t: (0, 0)),
            pl.BlockSpec((I, H), lambda t: (0, 0)),
            pl.BlockSpec((H, I), lambda t: (0, 0)),
        ],
        out_specs=[
            pl.BlockSpec((BT, H), lambda t: (t, 0)),
            pl.BlockSpec((BT, K), lambda t: (t, 0)),
            pl.BlockSpec((BT, K), lambda t: (t, 0)),
        ],
        out_shape=[
            jax.ShapeDtypeStruct((T, H), jnp.float32),
            jax.ShapeDtypeStruct((T, K), jnp.int32),
            jax.ShapeDtypeStruct((T, K), jnp.float32),
        ],
        compiler_params=pltpu.CompilerParams(vmem_limit_bytes=64 * 1024 * 1024),
    )(x, router_w, bias2d,
      _cast_bf16_2d(sh_gate_w, IS_, H), _cast_bf16_2d(sh_up_w, IS_, H),
      _cast_bf16_2d(sh_down_w, H, IS_))

    gate_b, up_b, down_b = _cast3_bf16(gate_w, up_w, down_w)

    eids = topk_idx.reshape(A)
    w3d = topk_w.reshape(NW, NG, GRP)

    mesh = plsc.VectorSubcoreMesh(core_axis_name="c", subcore_axis_name="s")
    sc_params = pltpu.CompilerParams(needs_layout_passes=False)
    dispatch = functools.partial(
        pl.kernel,
        mesh=mesh,
        compiler_params=sc_params,
        out_type=[
            jax.ShapeDtypeStruct((S, H), jnp.float32),
            jax.ShapeDtypeStruct((NW, NG, GRP), jnp.int32),
            jax.ShapeDtypeStruct((48,), jnp.int32),
        ],
        scratch_types=[
            pltpu.VMEM((A,), jnp.int32),
            pltpu.VMEM((NG, GRP), jnp.int32),
            pltpu.VMEM((NG, GRP), jnp.int32),
            pltpu.VMEM((GRP, H), jnp.float32),
            pltpu.VMEM((GRP, H), jnp.float32),
            pltpu.VMEM((48,), jnp.int32),
            pltpu.VMEM((16,), jnp.int32),
            pltpu.VMEM((16,), jnp.int32),
            pltpu.VMEM((16,), jnp.int32),
            pltpu.SemaphoreType.DMA,
            pltpu.SemaphoreType.DMA,
            pltpu.SemaphoreType.DMA,
            pltpu.SemaphoreType.DMA,
        ],
    )(_dispatch_body)
    x_sorted, pos, btbl = dispatch(eids, x)

    grid_spec = pltpu.PrefetchScalarGridSpec(
        num_scalar_prefetch=1,
        grid=(NB,),
        in_specs=[
            pl.BlockSpec((M, H), lambda b, s: (b, 0)),
            pl.BlockSpec((1, I, H), lambda b, s: (s[b], 0, 0)),
            pl.BlockSpec((1, I, H), lambda b, s: (s[b], 0, 0)),
            pl.BlockSpec((1, H, I), lambda b, s: (s[b], 0, 0)),
        ],
        out_specs=pl.BlockSpec((M, H), lambda b, s: (b, 0)),
    )
    y_sorted = pl.pallas_call(
        _ffn_block_body,
        grid_spec=grid_spec,
        out_shape=jax.ShapeDtypeStruct((S, H), jnp.float32),
    )(btbl, x_sorted, gate_b, up_b, down_b)

    combine = functools.partial(
        pl.kernel,
        mesh=mesh,
        compiler_params=sc_params,
        out_type=jax.ShapeDtypeStruct((T, H), jnp.float32),
        scratch_types=[
            pltpu.VMEM((NG, GRP), jnp.int32),
            pltpu.VMEM((NG, GRP), jnp.float32),
            pltpu.VMEM((GRP, H), jnp.float32),
            pltpu.VMEM((GRP, H), jnp.float32),
            pltpu.VMEM((GRP // 2, H), jnp.float32),
            pltpu.SemaphoreType.DMA,
            pltpu.SemaphoreType.DMA,
            pltpu.SemaphoreType.DMA,
        ],
    )(_combine_body)
    return combine(out_init, y_sorted, pos, w3d)


# speedup vs baseline: 1.0285x; 1.0007x over previous
import functools

import jax
import jax.numpy as jnp
from jax import lax
from jax.experimental import pallas as pl
from jax.experimental.pallas import tpu as pltpu
from jax.experimental.pallas import tpu_sc as plsc

T = 4096
H = 2048
I = 1408
E = 8
K = 2
A = T * K
M = 256
MSH = 8
NB = 40
S = NB * M
NW = 32
CHUNK = A // NW
IS_ = I
NG = 16
GRP = CHUNK // NG
BT = 1024


def _lane():
    return lax.iota(jnp.int32, 16)


def _cast_body(w_ref, o_ref):
    o_ref[...] = w_ref[...].astype(jnp.bfloat16)


def _cast3_body(g_ref, u_ref, d_ref, go_ref, uo_ref, do_ref):
    go_ref[...] = g_ref[...].astype(jnp.bfloat16)
    uo_ref[...] = u_ref[...].astype(jnp.bfloat16)
    do_ref[...] = d_ref[...].astype(jnp.bfloat16)


def _cast3_bf16(g, u, d):
    spec_ih = pl.BlockSpec((1, I // 2, H), lambda e, i: (e, i, 0))
    spec_hi = pl.BlockSpec((1, H // 2, I), lambda e, i: (e, i, 0))
    return pl.pallas_call(
        _cast3_body,
        grid=(E, 2),
        in_specs=[spec_ih, spec_ih, spec_hi],
        out_specs=[spec_ih, spec_ih, spec_hi],
        out_shape=[
            jax.ShapeDtypeStruct(g.shape, jnp.bfloat16),
            jax.ShapeDtypeStruct(u.shape, jnp.bfloat16),
            jax.ShapeDtypeStruct(d.shape, jnp.bfloat16),
        ],
    )(g, u, d)


def _cast_bf16_2d(w, second, minor):
    return pl.pallas_call(
        _cast_body,
        grid=(2,),
        in_specs=[pl.BlockSpec((second // 2, minor), lambda i: (i, 0))],
        out_specs=pl.BlockSpec((second // 2, minor), lambda i: (i, 0)),
        out_shape=jax.ShapeDtypeStruct(w.shape, jnp.bfloat16),
    )(w)


def _router_body(x_ref, rw_ref, bias_ref, idx_ref, w_ref):
    x = x_ref[...]
    logits = jnp.dot(x, rw_ref[...].T, preferred_element_type=jnp.float32)
    logits = logits + bias_ref[...]
    scores = jax.nn.sigmoid(logits)
    iota = lax.broadcasted_iota(jnp.int32, scores.shape, 1)
    w1 = jnp.max(scores, axis=-1, keepdims=True)
    i1 = jnp.argmax(scores, axis=-1)[:, None]
    masked = jnp.where(iota == i1, -jnp.inf, scores)
    w2 = jnp.max(masked, axis=-1, keepdims=True)
    i2 = jnp.argmax(masked, axis=-1)[:, None]
    denom = w1 + w2 + 1e-8
    idx_ref[...] = jnp.concatenate([i1, i2], axis=1)
    w_ref[...] = jnp.concatenate([w1 / denom, w2 / denom], axis=1)


def _shared_body(x_ref, g_ref, u_ref, d_ref, out_ref):
    xb = x_ref[...].astype(jnp.bfloat16)
    g = jnp.dot(xb, g_ref[...].T, preferred_element_type=jnp.float32)
    u = jnp.dot(xb, u_ref[...].T, preferred_element_type=jnp.float32)
    h = (jax.nn.silu(g) * u).astype(jnp.bfloat16)
    out_ref[...] = jnp.dot(h, d_ref[...].T, preferred_element_type=jnp.float32)


def _dispatch_body(eids_hbm, xb_hbm, xs_hbm, pos_hbm, btbl_hbm,
                   eids_v, posb, tokb, rows0, rows1, btbl_v,
                   hist_all, hist_pre, runh,
                   gsem0, gsem1, ssem0, ssem1):
    _LANE = _lane()
    wid = lax.axis_index("c") * 16 + lax.axis_index("s")
    my_vreg0 = wid * (CHUNK // 16)

    pltpu.sync_copy(eids_hbm, eids_v)

    lane_eq = [(_LANE == e) for e in range(E)]
    zeros16 = jnp.zeros((16,), jnp.int32)
    ones16 = jnp.full((16,), 1, jnp.int32)
    hist_all[...] = zeros16
    hist_pre[...] = zeros16

    def scan_step(i, carry):
        v = eids_v[pl.ds(i * 16, 16)]
        plsc.addupdate_scatter(hist_all, [v], ones16)
        pre = jnp.where(i < my_vreg0, 1, 0)
        plsc.addupdate_scatter(hist_pre, [v], ones16 * pre)
        return carry

    lax.fori_loop(0, A // 16, scan_step, 0)
    cnt_all = hist_all[...]
    cnt_pre = hist_pre[...]

    nb = (cnt_all + (M - 1)) >> MSH
    nbc_incl = plsc.cumsum(nb)
    pstart = (nbc_incl - nb) * M

    runh[...] = pstart + cnt_pre
    for j in range(CHUNK // 16):
        v = eids_v[pl.ds((my_vreg0 + j) * 16, 16)]
        base = plsc.load_gather(runh, [v])
        excl = zeros16
        for e in range(E):
            m = (v == e).astype(jnp.int32)
            cs = plsc.cumsum(m)
            excl = excl + (cs - m) * m
        plsc.addupdate_scatter(runh, [v], ones16)
        posb[j, :] = base + excl
        tokb[j, :] = (wid * CHUNK + j * 16 + _LANE) >> 1

    @pl.when(wid == 0)
    def _():
        nbc_s = [jnp.sum(jnp.where(lane_eq[e], nbc_incl, 0)) for e in range(E)]
        total = nbc_s[E - 1]
        for r in range(3):
            b = _LANE + r * 16
            acc = jnp.zeros((16,), jnp.int32)
            for e in range(E):
                lo = nbc_s[e - 1] if e > 0 else 0
                acc = acc + e * ((b >= lo) & (b < nbc_s[e])).astype(jnp.int32)
            btbl_v[pl.ds(r * 16, 16)] = jnp.where(b < total, acc, 0)
        pltpu.sync_copy(btbl_v, btbl_hbm)

    pltpu.sync_copy(posb, pos_hbm.at[wid])

    bufs = (rows0, rows1)
    gsems = (gsem0, gsem1)
    ssems = (ssem0, ssem1)
    scat = [None, None]

    def gather(g):
        return pltpu.async_copy(xb_hbm.at[tokb.at[g]], bufs[g % 2],
                                gsems[g % 2])

    cg = gather(0)
    for g in range(NG):
        cg.wait()
        scat[g % 2] = pltpu.async_copy(bufs[g % 2], xs_hbm.at[posb.at[g]],
                                       ssems[g % 2])
        if g + 1 < NG:
            if scat[(g + 1) % 2] is not None:
                scat[(g + 1) % 2].wait()
            cg = gather(g + 1)
    scat[0].wait()
    scat[1].wait()


def _ffn_block_body(s_ref, xs_ref, g_ref, u_ref, d_ref, y_ref):
    xb = xs_ref[...].astype(jnp.bfloat16)
    g = jnp.dot(xb, g_ref[0].T, preferred_element_type=jnp.float32)
    u = jnp.dot(xb, u_ref[0].T, preferred_element_type=jnp.float32)
    h = (jax.nn.silu(g) * u).astype(jnp.bfloat16)
    y_ref[...] = jnp.dot(h, d_ref[0].T, preferred_element_type=jnp.float32)


def _combine_body(init_hbm, y_hbm, pos_hbm, w_hbm, out_hbm,
                  posb, wb, y0, y1, shbuf, gsem0, gsem1, sem):
    _LANE = _lane()
    wid = lax.axis_index("c") * 16 + lax.axis_index("s")
    pltpu.sync_copy(pos_hbm.at[wid], posb)
    pltpu.sync_copy(w_hbm.at[wid], wb)
    ybufs = (y0, y1)
    gsems = (gsem0, gsem1)
    TPG = GRP // 2

    def gather(g):
        return pltpu.async_copy(y_hbm.at[posb.at[g]], ybufs[g % 2],
                                gsems[g % 2])

    cg = gather(0)
    for g in range(NG):
        tok_base = wid * (T // NW) + g * TPG
        pltpu.sync_copy(init_hbm.at[pl.ds(tok_base, TPG)], shbuf)
        wrow = wb[g, :]
        ws = []
        for a in range(GRP):
            ws.append(jnp.sum(jnp.where(_LANE == a, wrow, 0.0)))
        cg.wait()
        yb = ybufs[g % 2]
        if g + 1 < NG:
            cg = gather(g + 1)

        def vbody(i, _, yb=yb, ws=ws):
            sl = pl.ds(i * 16, 16)
            for t in range(TPG):
                shbuf[t, sl] = (shbuf[t, sl] + ws[2 * t] * yb[2 * t, sl]
                                + ws[2 * t + 1] * yb[2 * t + 1, sl])
            return 0

        lax.fori_loop(0, H // 16, vbody, 0)
        pltpu.sync_copy(shbuf, out_hbm.at[pl.ds(tok_base, TPG)])


def kernel(x, router_w, lb_bias, gate_w, up_w, down_w, sh_gate_w, sh_up_w, sh_down_w):
    bias2d = lb_bias.reshape(1, E)

    grid = (T // BT,)
    topk_idx, topk_w = pl.pallas_call(
        _router_body,
        grid=grid,
        in_specs=[
            pl.BlockSpec((BT, H), lambda t: (t, 0)),
            pl.BlockSpec((E, H), lambda t: (0, 0)),
            pl.BlockSpec((1, E), lambda t: (0, 0)),
        ],
        out_specs=[
            pl.BlockSpec((BT, K), lambda t: (t, 0)),
            pl.BlockSpec((BT, K), lambda t: (t, 0)),
        ],
        out_shape=[
            jax.ShapeDtypeStruct((T, K), jnp.int32),
            jax.ShapeDtypeStruct((T, K), jnp.float32),
        ],
    )(x, router_w, bias2d)

    out_init = pl.pallas_call(
        _shared_body,
        grid=grid,
        in_specs=[
            pl.BlockSpec((BT, H), lambda t: (t, 0)),
            pl.BlockSpec((I, H), lambda t: (0, 0)),
            pl.BlockSpec((I, H), lambda t: (0, 0)),
            pl.BlockSpec((H, I), lambda t: (0, 0)),
        ],
        out_specs=pl.BlockSpec((BT, H), lambda t: (t, 0)),
        out_shape=jax.ShapeDtypeStruct((T, H), jnp.float32),
        compiler_params=pltpu.CompilerParams(vmem_limit_bytes=64 * 1024 * 1024),
    )(x, _cast_bf16_2d(sh_gate_w, IS_, H), _cast_bf16_2d(sh_up_w, IS_, H),
      _cast_bf16_2d(sh_down_w, H, IS_))

    gate_b, up_b, down_b = _cast3_bf16(gate_w, up_w, down_w)

    eids = topk_idx.reshape(A)
    w3d = topk_w.reshape(NW, NG, GRP)

    mesh = plsc.VectorSubcoreMesh(core_axis_name="c", subcore_axis_name="s")
    sc_params = pltpu.CompilerParams(needs_layout_passes=False)
    dispatch = functools.partial(
        pl.kernel,
        mesh=mesh,
        compiler_params=sc_params,
        out_type=[
            jax.ShapeDtypeStruct((S, H), jnp.float32),
            jax.ShapeDtypeStruct((NW, NG, GRP), jnp.int32),
            jax.ShapeDtypeStruct((48,), jnp.int32),
        ],
        scratch_types=[
            pltpu.VMEM((A,), jnp.int32),
            pltpu.VMEM((NG, GRP), jnp.int32),
            pltpu.VMEM((NG, GRP), jnp.int32),
            pltpu.VMEM((GRP, H), jnp.float32),
            pltpu.VMEM((GRP, H), jnp.float32),
            pltpu.VMEM((48,), jnp.int32),
            pltpu.VMEM((16,), jnp.int32),
            pltpu.VMEM((16,), jnp.int32),
            pltpu.VMEM((16,), jnp.int32),
            pltpu.SemaphoreType.DMA,
            pltpu.SemaphoreType.DMA,
            pltpu.SemaphoreType.DMA,
            pltpu.SemaphoreType.DMA,
        ],
    )(_dispatch_body)
    x_sorted, pos, btbl = dispatch(eids, x)

    grid_spec = pltpu.PrefetchScalarGridSpec(
        num_scalar_prefetch=1,
        grid=(NB,),
        in_specs=[
            pl.BlockSpec((M, H), lambda b, s: (b, 0)),
            pl.BlockSpec((1, I, H), lambda b, s: (s[b], 0, 0)),
            pl.BlockSpec((1, I, H), lambda b, s: (s[b], 0, 0)),
            pl.BlockSpec((1, H, I), lambda b, s: (s[b], 0, 0)),
        ],
        out_specs=pl.BlockSpec((M, H), lambda b, s: (b, 0)),
    )
    y_sorted = pl.pallas_call(
        _ffn_block_body,
        grid_spec=grid_spec,
        out_shape=jax.ShapeDtypeStruct((S, H), jnp.float32),
    )(btbl, x_sorted, gate_b, up_b, down_b)

    combine = functools.partial(
        pl.kernel,
        mesh=mesh,
        compiler_params=sc_params,
        out_type=jax.ShapeDtypeStruct((T, H), jnp.float32),
        scratch_types=[
            pltpu.VMEM((NG, GRP), jnp.int32),
            pltpu.VMEM((NG, GRP), jnp.float32),
            pltpu.VMEM((GRP, H), jnp.float32),
            pltpu.VMEM((GRP, H), jnp.float32),
            pltpu.VMEM((GRP // 2, H), jnp.float32),
            pltpu.SemaphoreType.DMA,
            pltpu.SemaphoreType.DMA,
            pltpu.SemaphoreType.DMA,
        ],
    )(_combine_body)
    return combine(out_init, y_sorted, pos, w3d)
